# baseline (speedup 1.0000x reference)
import functools

import jax
import jax.numpy as jnp
from jax import lax
from jax.experimental import pallas as pl
from jax.experimental.pallas import tpu as pltpu
from jax.experimental.pallas import tpu_sc as plsc

NE = 8
TOPK = 2
NG = 4
DM = 2048
DFF = 1408
T = 2048
NPAIR = T * TOPK
RB = 256
P = NPAIR + 7 * RB
NB = P // RB
BF = 128
NFF = DFF // BF
DSH = 2 * DFF
BFS = 256
NFS = DSH // BFS
TB = 256

@functools.cache
def _mesh():
    return plsc.VectorSubcoreMesh(core_axis_name="c", subcore_axis_name="s")


NTILES = 32
RPT = P // NTILES
TPT = T // NTILES


def _meta_body(ti_ref, pos_ref, meta_ref):
    ti = ti_ref[...]
    i1 = ti[:, 0:1]
    i2 = ti[:, 1:2]
    cols = lax.broadcasted_iota(jnp.int32, (T, NE), 1)
    sel = (jnp.where(cols == i1, 1.0, 0.0)
           + jnp.where(cols == i2, 1.0, 0.0))

    rr = lax.broadcasted_iota(jnp.int32, (T, T), 0)
    cc = lax.broadcasted_iota(jnp.int32, (T, T), 1)
    tri = jnp.where(rr > cc, 1.0, 0.0)
    pim = lax.dot_general(
        tri, sel, (((1,), (0,)), ((), ())),
        preferred_element_type=jnp.float32, precision=lax.Precision.HIGHEST)

    counts = jnp.sum(sel, axis=0, keepdims=True)
    padded = jnp.ceil(counts * (1.0 / RB)) * RB
    r8 = lax.broadcasted_iota(jnp.int32, (NE, NE), 0)
    c8 = lax.broadcasted_iota(jnp.int32, (NE, NE), 1)
    m_lt = jnp.where(r8 < c8, 1.0, 0.0)
    m_le = jnp.where(r8 <= c8, 1.0, 0.0)
    s_excl = lax.dot_general(
        padded, m_lt, (((1,), (0,)), ((), ())),
        preferred_element_type=jnp.float32, precision=lax.Precision.HIGHEST)
    s_incl = lax.dot_general(
        padded, m_le, (((1,), (0,)), ((), ())),
        preferred_element_type=jnp.float32, precision=lax.Precision.HIGHEST)
    total = jnp.max(s_incl)
    nv = (total * (1.0 / RB)).astype(jnp.int32)

    posmat = s_excl + pim
    p1 = jnp.sum(jnp.where(cols == i1, posmat, 0.0), axis=1, keepdims=True)
    p2 = jnp.sum(jnp.where(cols == i2, posmat, 0.0), axis=1, keepdims=True)
    pos_ref[...] = jnp.concatenate([p1, p2], axis=1).astype(jnp.int32)

    s_col = jnp.sum(
        jnp.where(r8 == c8, 1.0, 0.0) * s_incl, axis=1, keepdims=True)
    bstart = (lax.broadcasted_iota(jnp.int32, (NE, 32), 1) * RB).astype(
        jnp.float32)
    be = jnp.sum(jnp.where(s_col <= bstart, 1, 0), axis=0, keepdims=True)
    be = jnp.minimum(be, NE - 1)
    l32 = lax.broadcasted_iota(jnp.int32, (1, 32), 1)
    meta_ref[...] = (jnp.where(l32 < NB, be, 0)
                     + jnp.where(l32 == NB, nv, 0))


def _meta(ti):
    return pl.pallas_call(
        _meta_body,
        out_shape=(
            jax.ShapeDtypeStruct((T, TOPK), jnp.int32),
            jax.ShapeDtypeStruct((1, 32), jnp.int32),
        ),
    )(ti)


def _dispatch_body(pp_hbm, pw_hbm, st_hbm, sw_hbm, ppv, pwv, stv, swv):
    cid = lax.axis_index("c")
    sid = lax.axis_index("s")

    @pl.when(jnp.logical_and(cid == 0, sid == 0))
    def _():
        pltpu.sync_copy(pp_hbm, ppv)
        pltpu.sync_copy(pw_hbm, pwv)

        def ini(k, carry):
            stv[pl.ds(k * 16, 16)] = jnp.zeros((16,), jnp.int32)
            swv[pl.ds(k * 16, 16)] = jnp.zeros((16,), jnp.float32)
            return carry

        lax.fori_loop(0, P // 16, ini, 0)

        def scat(k, carry):
            idx = ppv[pl.ds(k * 16, 16)]
            tok = lax.shift_right_logical(
                lax.iota(jnp.int32, 16) + k * 16, 1)
            plsc.store_scatter(stv, [idx], tok)
            w = pwv[pl.ds(k * 16, 16)]
            plsc.store_scatter(swv, [idx], w)
            return carry

        lax.fori_loop(0, NPAIR // 16, scat, 0)
        pltpu.sync_copy(stv, st_hbm)
        pltpu.sync_copy(swv, sw_hbm)


def _dispatch(pp_flat, pw_flat):
    return pl.kernel(
        _dispatch_body,
        out_type=(
            jax.ShapeDtypeStruct((P,), jnp.int32),
            jax.ShapeDtypeStruct((P,), jnp.float32),
        ),
        mesh=_mesh(),
        scratch_types=[
            pltpu.VMEM((NPAIR,), jnp.int32),
            pltpu.VMEM((NPAIR,), jnp.float32),
            pltpu.VMEM((P,), jnp.int32),
            pltpu.VMEM((P,), jnp.float32),
        ],
        compiler_params=pltpu.CompilerParams(needs_layout_passes=False),
    )(pp_flat, pw_flat)


_GCH = 24
_GOFFS = [24 * i for i in range(7)] + [168]
_GSIZES = [24] * 7 + [16]


def _gather_body(st_hbm, x_hbm, out_hbm, idxv, b0, b1, gs0, gs1, ws0, ws1):
    wid = lax.axis_index("s") * 2 + lax.axis_index("c")
    base = wid * RPT
    pltpu.sync_copy(st_hbm.at[pl.ds(base, RPT)], idxv)
    bufs = (b0, b1)
    gsems = (gs0, gs1)
    wsems = (ws0, ws1)
    n = len(_GOFFS)
    gd = [None, None]
    wd = [None, None]
    for ci in range(n + 1):
        if ci < n:
            sz = _GSIZES[ci]
            if wd[ci % 2] is not None:
                wd[ci % 2].wait()
            gd[ci % 2] = pltpu.async_copy(
                x_hbm.at[idxv.at[pl.ds(_GOFFS[ci], sz)]],
                bufs[ci % 2].at[pl.ds(0, sz)], gsems[ci % 2])
        if ci > 0:
            pj = ci - 1
            szp = _GSIZES[pj]
            gd[pj % 2].wait()
            wd[pj % 2] = pltpu.async_copy(
                bufs[pj % 2].at[pl.ds(0, szp)],
                out_hbm.at[pl.ds(base + _GOFFS[pj], szp)], wsems[pj % 2])
    wd[0].wait()
    wd[1].wait()


def _gather(st, x):
    return pl.kernel(
        _gather_body,
        out_type=jax.ShapeDtypeStruct((P, DM), jnp.float32),
        mesh=_mesh(),
        scratch_types=[
            pltpu.VMEM((RPT,), jnp.int32),
            pltpu.VMEM((_GCH, DM), jnp.float32),
            pltpu.VMEM((_GCH, DM), jnp.float32),
            pltpu.SemaphoreType.DMA,
            pltpu.SemaphoreType.DMA,
            pltpu.SemaphoreType.DMA,
            pltpu.SemaphoreType.DMA,
        ],
    )(st, x)


def _ffn_body(be_ref, nv_ref, x_ref, g_ref, u_ref, d_ref, w_ref, o_ref, acc):
    i = pl.program_id(0)
    j = pl.program_id(1)

    @pl.when(i < nv_ref[0])
    def _():
        x = x_ref[...]
        g = lax.dot_general(x, g_ref[0], (((1,), (1,)), ((), ())),
                            preferred_element_type=jnp.float32)
        u = lax.dot_general(x, u_ref[0], (((1,), (1,)), ((), ())),
                            preferred_element_type=jnp.float32)
        h = g * jax.nn.sigmoid(g) * u
        part = lax.dot_general(h, d_ref[0], (((1,), (1,)), ((), ())),
                               preferred_element_type=jnp.float32)

        @pl.when(j == 0)
        def _():
            acc[...] = part

        @pl.when(j > 0)
        def _():
            acc[...] += part

        @pl.when(j == NFF - 1)
        def _():
            o_ref[...] = acc[...] * w_ref[...]


def _ffn(be, nv, xs, egu, ed, sw_col):
    grid_spec = pltpu.PrefetchScalarGridSpec(
        num_scalar_prefetch=2,
        grid=(NB, NFF),
        in_specs=[
            pl.BlockSpec((RB, DM), lambda i, j, be, nv: (i, 0)),
            pl.BlockSpec((1, BF, DM), lambda i, j, be, nv: (be[i], j, 0)),
            pl.BlockSpec((1, BF, DM), lambda i, j, be, nv: (be[i], j + NFF, 0)),
            pl.BlockSpec((1, DM, BF), lambda i, j, be, nv: (be[i], 0, j)),
            pl.BlockSpec((RB, 1), lambda i, j, be, nv: (i, 0)),
        ],
        out_specs=pl.BlockSpec((RB, DM), lambda i, j, be, nv: (i, 0)),
        scratch_shapes=[pltpu.VMEM((RB, DM), jnp.float32)],
    )
    return pl.pallas_call(
        _ffn_body,
        grid_spec=grid_spec,
        out_shape=jax.ShapeDtypeStruct((P, DM), jnp.float32),
        compiler_params=pltpu.CompilerParams(
            dimension_semantics=("arbitrary", "arbitrary")),
    )(be, nv, xs, egu, egu, ed, sw_col)


def _shared_body(x_ref, g_ref, u_ref, d_ref, o_ref, acc):
    j = pl.program_id(1)
    x = x_ref[...]
    g = lax.dot_general(x, g_ref[...], (((1,), (1,)), ((), ())),
                        preferred_element_type=jnp.float32)
    u = lax.dot_general(x, u_ref[...], (((1,), (1,)), ((), ())),
                        preferred_element_type=jnp.float32)
    h = g * jax.nn.sigmoid(g) * u
    part = lax.dot_general(h, d_ref[...], (((1,), (1,)), ((), ())),
                           preferred_element_type=jnp.float32)

    @pl.when(j == 0)
    def _():
        acc[...] = part

    @pl.when(j > 0)
    def _():
        acc[...] += part

    @pl.when(j == NFS - 1)
    def _():
        o_ref[...] = acc[...]


def _shared(x, sgu, sd):
    return pl.pallas_call(
        _shared_body,
        grid=(T // TB, NFS),
        in_specs=[
            pl.BlockSpec((TB, DM), lambda i, j: (i, 0)),
            pl.BlockSpec((BFS, DM), lambda i, j: (j, 0)),
            pl.BlockSpec((BFS, DM), lambda i, j: (j + NFS, 0)),
            pl.BlockSpec((DM, BFS), lambda i, j: (0, j)),
        ],
        out_specs=pl.BlockSpec((TB, DM), lambda i, j: (i, 0)),
        out_shape=jax.ShapeDtypeStruct((T, DM), jnp.float32),
        scratch_shapes=[pltpu.VMEM((TB, DM), jnp.float32)],
        compiler_params=pltpu.CompilerParams(
            dimension_semantics=("arbitrary", "arbitrary")),
    )(x, sgu, sgu, sd)


_CCH = 8
_CNC = TPT // _CCH


def _combine_body(y_hbm, p0_hbm, p1_hbm, sh_hbm, out_hbm,
                  i0v, i1v, y0a, y0b, y1a, y1b, shb, ob,
                  s0a, s0b, s1a, s1b):
    wid = lax.axis_index("s") * 2 + lax.axis_index("c")
    tb = wid * TPT
    pltpu.sync_copy(p0_hbm.at[pl.ds(tb, TPT)], i0v)
    pltpu.sync_copy(p1_hbm.at[pl.ds(tb, TPT)], i1v)
    y0 = (y0a, y0b)
    y1 = (y1a, y1b)
    s0 = (s0a, s0b)
    s1 = (s1a, s1b)
    gd0 = [None, None]
    gd1 = [None, None]
    for c in range(_CNC + 1):
        if c < _CNC:
            sl = pl.ds(c * _CCH, _CCH)
            gd0[c % 2] = pltpu.async_copy(y_hbm.at[i0v.at[sl]],
                                          y0[c % 2], s0[c % 2])
            gd1[c % 2] = pltpu.async_copy(y_hbm.at[i1v.at[sl]],
                                          y1[c % 2], s1[c % 2])
        if c > 0:
            p = c - 1
            pltpu.sync_copy(sh_hbm.at[pl.ds(tb + p * _CCH, _CCH)], shb)
            gd0[p % 2].wait()
            gd1[p % 2].wait()
            for t in range(_CCH):
                def addk(k, carry):
                    sl2 = pl.ds(k * 16, 16)
                    ob[t, sl2] = (y0[p % 2][t, sl2] + y1[p % 2][t, sl2]
                                  + shb[t, sl2])
                    return carry

                lax.fori_loop(0, DM // 16, addk, 0)
            pltpu.sync_copy(ob, out_hbm.at[pl.ds(tb + p * _CCH, _CCH)])


def _combine(ys, p0, p1, sh):
    return pl.kernel(
        _combine_body,
        out_type=jax.ShapeDtypeStruct((T, DM), jnp.float32),
        mesh=_mesh(),
        scratch_types=[
            pltpu.VMEM((TPT,), jnp.int32),
            pltpu.VMEM((TPT,), jnp.int32),
            pltpu.VMEM((_CCH, DM), jnp.float32),
            pltpu.VMEM((_CCH, DM), jnp.float32),
            pltpu.VMEM((_CCH, DM), jnp.float32),
            pltpu.VMEM((_CCH, DM), jnp.float32),
            pltpu.VMEM((_CCH, DM), jnp.float32),
            pltpu.VMEM((_CCH, DM), jnp.float32),
            pltpu.SemaphoreType.DMA,
            pltpu.SemaphoreType.DMA,
            pltpu.SemaphoreType.DMA,
            pltpu.SemaphoreType.DMA,
        ],
    )(ys, p0, p1, sh)


def kernel(hidden_states, gate_w, expert_gate_up, expert_down,
           shared_gate_up, shared_down):
    router_logits = hidden_states @ gate_w.T
    scores = jax.nn.softmax(router_logits.astype(jnp.float32), axis=-1)
    group_scores = jnp.max(scores.reshape(T, 2, NG), axis=-1)
    _, group_idx = jax.lax.top_k(group_scores, 1)
    group_mask = jnp.zeros((T, 2), dtype=scores.dtype).at[
        jnp.arange(T)[:, None], group_idx].set(1.0)
    score_mask = jnp.repeat(group_mask, NG, axis=1)
    masked_scores = jnp.where(score_mask > 0, scores, 0.0)
    topk_w, topk_idx = jax.lax.top_k(masked_scores, TOPK)
    pair_w = topk_w / (jnp.sum(topk_w, axis=-1, keepdims=True) + 1e-20)

    pair_pos, meta = _meta(topk_idx)
    be = meta[0, :NB]
    nv = meta[0, NB:NB + 1]
    st, sw = _dispatch(pair_pos.reshape(NPAIR), pair_w.reshape(NPAIR))
    sh = _shared(hidden_states, shared_gate_up, shared_down)
    xs = _gather(st, hidden_states)
    ys = _ffn(be, nv, xs, expert_gate_up, expert_down, sw.reshape(P, 1))
    return _combine(ys, pair_pos[:, 0], pair_pos[:, 1], sh)


# speedup vs baseline: 1.1039x; 1.1039x over previous
import functools

import jax
import jax.numpy as jnp
from jax import lax
from jax.experimental import pallas as pl
from jax.experimental.pallas import tpu as pltpu
from jax.experimental.pallas import tpu_sc as plsc

NE = 8
TOPK = 2
NG = 4
DM = 2048
DFF = 1408
T = 2048
NPAIR = T * TOPK
RB = 256
P = NPAIR + 7 * RB
NB = P // RB
BF = 128
NFF = DFF // BF
DSH = 2 * DFF
BFS = 256
NFS = DSH // BFS
TB = 256

@functools.cache
def _mesh():
    return plsc.VectorSubcoreMesh(core_axis_name="c", subcore_axis_name="s")


NTILES = 32
RPT = P // NTILES
TPT = T // NTILES


def _meta_body(ti_ref, pos_ref, meta_ref):
    ti = ti_ref[...]
    i1 = ti[:, 0:1]
    i2 = ti[:, 1:2]
    cols = lax.broadcasted_iota(jnp.int32, (T, NE), 1)
    sel = (jnp.where(cols == i1, 1.0, 0.0)
           + jnp.where(cols == i2, 1.0, 0.0))

    rr = lax.broadcasted_iota(jnp.int32, (T, T), 0)
    cc = lax.broadcasted_iota(jnp.int32, (T, T), 1)
    tri = jnp.where(rr > cc, 1.0, 0.0)
    pim = lax.dot_general(
        tri, sel, (((1,), (0,)), ((), ())),
        preferred_element_type=jnp.float32, precision=lax.Precision.HIGHEST)

    counts = jnp.sum(sel, axis=0, keepdims=True)
    padded = jnp.ceil(counts * (1.0 / RB)) * RB
    r8 = lax.broadcasted_iota(jnp.int32, (NE, NE), 0)
    c8 = lax.broadcasted_iota(jnp.int32, (NE, NE), 1)
    m_lt = jnp.where(r8 < c8, 1.0, 0.0)
    m_le = jnp.where(r8 <= c8, 1.0, 0.0)
    s_excl = lax.dot_general(
        padded, m_lt, (((1,), (0,)), ((), ())),
        preferred_element_type=jnp.float32, precision=lax.Precision.HIGHEST)
    s_incl = lax.dot_general(
        padded, m_le, (((1,), (0,)), ((), ())),
        preferred_element_type=jnp.float32, precision=lax.Precision.HIGHEST)
    total = jnp.max(s_incl)
    nv = (total * (1.0 / RB)).astype(jnp.int32)

    posmat = s_excl + pim
    p1 = jnp.sum(jnp.where(cols == i1, posmat, 0.0), axis=1, keepdims=True)
    p2 = jnp.sum(jnp.where(cols == i2, posmat, 0.0), axis=1, keepdims=True)
    pos_ref[...] = jnp.concatenate([p1, p2], axis=1).astype(jnp.int32)

    s_col = jnp.sum(
        jnp.where(r8 == c8, 1.0, 0.0) * s_incl, axis=1, keepdims=True)
    bstart = (lax.broadcasted_iota(jnp.int32, (NE, 32), 1) * RB).astype(
        jnp.float32)
    be = jnp.sum(jnp.where(s_col <= bstart, 1, 0), axis=0, keepdims=True)
    be = jnp.minimum(be, NE - 1)
    l32 = lax.broadcasted_iota(jnp.int32, (1, 32), 1)
    meta_ref[...] = (jnp.where(l32 < NB, be, 0)
                     + jnp.where(l32 == NB, nv, 0))


def _meta(ti):
    return pl.pallas_call(
        _meta_body,
        out_shape=(
            jax.ShapeDtypeStruct((T, TOPK), jnp.int32),
            jax.ShapeDtypeStruct((1, 32), jnp.int32),
        ),
    )(ti)


def _dispatch_body(pp_hbm, pw_hbm, st_hbm, sw_hbm, ppv, pwv, stv, swv):
    cid = lax.axis_index("c")
    sid = lax.axis_index("s")

    @pl.when(jnp.logical_and(cid == 0, sid == 0))
    def _():
        pltpu.sync_copy(pp_hbm, ppv)
        pltpu.sync_copy(pw_hbm, pwv)

        def ini(k, carry):
            stv[pl.ds(k * 16, 16)] = jnp.zeros((16,), jnp.int32)
            swv[pl.ds(k * 16, 16)] = jnp.zeros((16,), jnp.float32)
            return carry

        lax.fori_loop(0, P // 16, ini, 0)

        def scat(k, carry):
            idx = ppv[pl.ds(k * 16, 16)]
            tok = lax.shift_right_logical(
                lax.iota(jnp.int32, 16) + k * 16, 1)
            plsc.store_scatter(stv, [idx], tok)
            w = pwv[pl.ds(k * 16, 16)]
            plsc.store_scatter(swv, [idx], w)
            return carry

        lax.fori_loop(0, NPAIR // 16, scat, 0)
        pltpu.sync_copy(stv, st_hbm)
        pltpu.sync_copy(swv, sw_hbm)


def _dispatch(pp_flat, pw_flat):
    return pl.kernel(
        _dispatch_body,
        out_type=(
            jax.ShapeDtypeStruct((P,), jnp.int32),
            jax.ShapeDtypeStruct((P,), jnp.float32),
        ),
        mesh=_mesh(),
        scratch_types=[
            pltpu.VMEM((NPAIR,), jnp.int32),
            pltpu.VMEM((NPAIR,), jnp.float32),
            pltpu.VMEM((P,), jnp.int32),
            pltpu.VMEM((P,), jnp.float32),
        ],
        compiler_params=pltpu.CompilerParams(needs_layout_passes=False),
    )(pp_flat, pw_flat)


_GCH = 24
_GOFFS = [24 * i for i in range(7)] + [168]
_GSIZES = [24] * 7 + [16]


def _gather_body(st_hbm, x_hbm, out_hbm, idxv, b0, b1, gs0, gs1, ws0, ws1):
    wid = lax.axis_index("s") * 2 + lax.axis_index("c")
    base = wid * RPT
    pltpu.sync_copy(st_hbm.at[pl.ds(base, RPT)], idxv)
    bufs = (b0, b1)
    gsems = (gs0, gs1)
    wsems = (ws0, ws1)
    n = len(_GOFFS)
    gd = [None, None]
    wd = [None, None]
    for ci in range(n + 1):
        if ci < n:
            sz = _GSIZES[ci]
            if wd[ci % 2] is not None:
                wd[ci % 2].wait()
            gd[ci % 2] = pltpu.async_copy(
                x_hbm.at[idxv.at[pl.ds(_GOFFS[ci], sz)]],
                bufs[ci % 2].at[pl.ds(0, sz)], gsems[ci % 2])
        if ci > 0:
            pj = ci - 1
            szp = _GSIZES[pj]
            gd[pj % 2].wait()
            wd[pj % 2] = pltpu.async_copy(
                bufs[pj % 2].at[pl.ds(0, szp)],
                out_hbm.at[pl.ds(base + _GOFFS[pj], szp)], wsems[pj % 2])
    wd[0].wait()
    wd[1].wait()


def _gather(st, x):
    return pl.kernel(
        _gather_body,
        out_type=jax.ShapeDtypeStruct((P, DM), jnp.float32),
        mesh=_mesh(),
        scratch_types=[
            pltpu.VMEM((RPT,), jnp.int32),
            pltpu.VMEM((_GCH, DM), jnp.float32),
            pltpu.VMEM((_GCH, DM), jnp.float32),
            pltpu.SemaphoreType.DMA,
            pltpu.SemaphoreType.DMA,
            pltpu.SemaphoreType.DMA,
            pltpu.SemaphoreType.DMA,
        ],
    )(st, x)


def _ffn_body(be_ref, nv_ref, x_ref, g_ref, u_ref, d_ref, w_ref, o_ref, hbuf):
    i = pl.program_id(0)
    j = pl.program_id(1)

    @pl.when(i < nv_ref[0])
    def _():
        x = x_ref[...]
        g = lax.dot_general(x, g_ref[0], (((1,), (1,)), ((), ())),
                            preferred_element_type=jnp.float32)
        u = lax.dot_general(x, u_ref[0], (((1,), (1,)), ((), ())),
                            preferred_element_type=jnp.float32)
        hbuf[:, pl.ds(j * BF, BF)] = g * jax.nn.sigmoid(g) * u

        @pl.when(j == NFF - 1)
        def _():
            o_ref[...] = lax.dot_general(
                hbuf[...], d_ref[0], (((1,), (1,)), ((), ())),
                preferred_element_type=jnp.float32) * w_ref[...]


def _ffn(be, nv, xs, egu, ed, sw_col):
    grid_spec = pltpu.PrefetchScalarGridSpec(
        num_scalar_prefetch=2,
        grid=(NB, NFF),
        in_specs=[
            pl.BlockSpec((RB, DM), lambda i, j, be, nv: (i, 0)),
            pl.BlockSpec((1, BF, DM), lambda i, j, be, nv: (be[i], j, 0)),
            pl.BlockSpec((1, BF, DM), lambda i, j, be, nv: (be[i], j + NFF, 0)),
            pl.BlockSpec((1, DM, DFF), lambda i, j, be, nv: (be[i], 0, 0)),
            pl.BlockSpec((RB, 1), lambda i, j, be, nv: (i, 0)),
        ],
        out_specs=pl.BlockSpec((RB, DM), lambda i, j, be, nv: (i, 0)),
        scratch_shapes=[pltpu.VMEM((RB, DFF), jnp.float32)],
    )
    return pl.pallas_call(
        _ffn_body,
        grid_spec=grid_spec,
        out_shape=jax.ShapeDtypeStruct((P, DM), jnp.float32),
        compiler_params=pltpu.CompilerParams(
            dimension_semantics=("arbitrary", "arbitrary")),
    )(be, nv, xs, egu, egu, ed, sw_col)


def _shared_body(x_ref, g_ref, u_ref, d_ref, o_ref, hbuf):
    j = pl.program_id(1)
    x = x_ref[...]
    g = lax.dot_general(x, g_ref[...], (((1,), (1,)), ((), ())),
                        preferred_element_type=jnp.float32)
    u = lax.dot_general(x, u_ref[...], (((1,), (1,)), ((), ())),
                        preferred_element_type=jnp.float32)
    hbuf[:, pl.ds(j * BFS, BFS)] = g * jax.nn.sigmoid(g) * u

    @pl.when(j == NFS - 1)
    def _():
        o_ref[...] = lax.dot_general(
            hbuf[...], d_ref[...], (((1,), (1,)), ((), ())),
            preferred_element_type=jnp.float32)


def _shared(x, sgu, sd):
    return pl.pallas_call(
        _shared_body,
        grid=(T // TB, NFS),
        in_specs=[
            pl.BlockSpec((TB, DM), lambda i, j: (i, 0)),
            pl.BlockSpec((BFS, DM), lambda i, j: (j, 0)),
            pl.BlockSpec((BFS, DM), lambda i, j: (j + NFS, 0)),
            pl.BlockSpec((DM, DSH), lambda i, j: (0, 0)),
        ],
        out_specs=pl.BlockSpec((TB, DM), lambda i, j: (i, 0)),
        out_shape=jax.ShapeDtypeStruct((T, DM), jnp.float32),
        scratch_shapes=[pltpu.VMEM((TB, DSH), jnp.float32)],
        compiler_params=pltpu.CompilerParams(
            dimension_semantics=("arbitrary", "arbitrary")),
    )(x, sgu, sgu, sd)


_CCH = 8
_CNC = TPT // _CCH


def _combine_body(y_hbm, p0_hbm, p1_hbm, sh_hbm, out_hbm,
                  i0v, i1v, y0a, y0b, y1a, y1b, shb, ob,
                  s0a, s0b, s1a, s1b):
    wid = lax.axis_index("s") * 2 + lax.axis_index("c")
    tb = wid * TPT
    pltpu.sync_copy(p0_hbm.at[pl.ds(tb, TPT)], i0v)
    pltpu.sync_copy(p1_hbm.at[pl.ds(tb, TPT)], i1v)
    y0 = (y0a, y0b)
    y1 = (y1a, y1b)
    s0 = (s0a, s0b)
    s1 = (s1a, s1b)
    gd0 = [None, None]
    gd1 = [None, None]
    for c in range(_CNC + 1):
        if c < _CNC:
            sl = pl.ds(c * _CCH, _CCH)
            gd0[c % 2] = pltpu.async_copy(y_hbm.at[i0v.at[sl]],
                                          y0[c % 2], s0[c % 2])
            gd1[c % 2] = pltpu.async_copy(y_hbm.at[i1v.at[sl]],
                                          y1[c % 2], s1[c % 2])
        if c > 0:
            p = c - 1
            pltpu.sync_copy(sh_hbm.at[pl.ds(tb + p * _CCH, _CCH)], shb)
            gd0[p % 2].wait()
            gd1[p % 2].wait()
            for t in range(_CCH):
                def addk(k, carry):
                    sl2 = pl.ds(k * 16, 16)
                    ob[t, sl2] = (y0[p % 2][t, sl2] + y1[p % 2][t, sl2]
                                  + shb[t, sl2])
                    return carry

                lax.fori_loop(0, DM // 16, addk, 0)
            pltpu.sync_copy(ob, out_hbm.at[pl.ds(tb + p * _CCH, _CCH)])


def _combine(ys, p0, p1, sh):
    return pl.kernel(
        _combine_body,
        out_type=jax.ShapeDtypeStruct((T, DM), jnp.float32),
        mesh=_mesh(),
        scratch_types=[
            pltpu.VMEM((TPT,), jnp.int32),
            pltpu.VMEM((TPT,), jnp.int32),
            pltpu.VMEM((_CCH, DM), jnp.float32),
            pltpu.VMEM((_CCH, DM), jnp.float32),
            pltpu.VMEM((_CCH, DM), jnp.float32),
            pltpu.VMEM((_CCH, DM), jnp.float32),
            pltpu.VMEM((_CCH, DM), jnp.float32),
            pltpu.VMEM((_CCH, DM), jnp.float32),
            pltpu.SemaphoreType.DMA,
            pltpu.SemaphoreType.DMA,
            pltpu.SemaphoreType.DMA,
            pltpu.SemaphoreType.DMA,
        ],
    )(ys, p0, p1, sh)


def kernel(hidden_states, gate_w, expert_gate_up, expert_down,
           shared_gate_up, shared_down):
    router_logits = hidden_states @ gate_w.T
    scores = jax.nn.softmax(router_logits.astype(jnp.float32), axis=-1)
    group_scores = jnp.max(scores.reshape(T, 2, NG), axis=-1)
    _, group_idx = jax.lax.top_k(group_scores, 1)
    group_mask = jnp.zeros((T, 2), dtype=scores.dtype).at[
        jnp.arange(T)[:, None], group_idx].set(1.0)
    score_mask = jnp.repeat(group_mask, NG, axis=1)
    masked_scores = jnp.where(score_mask > 0, scores, 0.0)
    topk_w, topk_idx = jax.lax.top_k(masked_scores, TOPK)
    pair_w = topk_w / (jnp.sum(topk_w, axis=-1, keepdims=True) + 1e-20)

    pair_pos, meta = _meta(topk_idx)
    be = meta[0, :NB]
    nv = meta[0, NB:NB + 1]
    st, sw = _dispatch(pair_pos.reshape(NPAIR), pair_w.reshape(NPAIR))
    sh = _shared(hidden_states, shared_gate_up, shared_down)
    xs = _gather(st, hidden_states)
    ys = _ffn(be, nv, xs, expert_gate_up, expert_down, sw.reshape(P, 1))
    return _combine(ys, pair_pos[:, 0], pair_pos[:, 1], sh)
